# baseline (speedup 1.0000x reference)
import functools

import jax
import jax.numpy as jnp
from jax import lax
from jax.experimental import pallas as pl
from jax.experimental.pallas import tpu as pltpu
from jax.experimental.pallas import tpu_sc as plsc

NC = 2
NS = 16
LANES = 16
CHUNK = 128
GCHUNK = 64
ROWBLK = 1024


def _round_up(v, m):
    return (v + m - 1) // m * m


def _make_gather_deg(F, npad, e2p, dw):
    cpt = e2p // (NC * NS * CHUNK)
    rpt = npad // (NC * NS)
    gct = rpt // GCHUNK
    slab = npad // NS

    mesh = plsc.VectorSubcoreMesh(core_axis_name="c", subcore_axis_name="s")

    @functools.partial(
        pl.kernel,
        out_type=(
            jax.ShapeDtypeStruct((npad, F), jnp.float32),
            jax.ShapeDtypeStruct((NC, npad, dw), jnp.float32),
        ),
        mesh=mesh,
        scratch_types=[
            pltpu.VMEM((rpt // GCHUNK, GCHUNK), jnp.int32),
            pltpu.VMEM((GCHUNK, F), jnp.float32),
            pltpu.VMEM((GCHUNK, F), jnp.float32),
            pltpu.VMEM((CHUNK,), jnp.int32),
            pltpu.VMEM((CHUNK,), jnp.int32),
            pltpu.VMEM((CHUNK,), jnp.float32),
            pltpu.VMEM((CHUNK,), jnp.float32),
            pltpu.VMEM((CHUNK, dw), jnp.float32),
            pltpu.VMEM_SHARED((npad, dw), jnp.float32),
            pltpu.SemaphoreType.DMA,
            pltpu.SemaphoreType.DMA,
            pltpu.SemaphoreType.DMA,
            pltpu.SemaphoreType.DMA,
        ],
    )
    def k1(emb_hbm, x_hbm, col_hbm, w_hbm, h0_hbm, deg_hbm,
           xb_v, grow0_v, grow1_v, ci0_v, ci1_v, wv0_v, wv1_v, wsrc_v, deg_sh,
           sem0, sem1, dsem0, dsem1):
        cid = lax.axis_index("c")
        sid = lax.axis_index("s")
        wid = cid * NS + sid
        ci = (ci0_v, ci1_v)
        wv = (wv0_v, wv1_v)
        dsem = (dsem0, dsem1)
        ebase = wid * cpt * CHUNK

        def didx_issue(g, q):
            off = ebase + g * CHUNK
            pltpu.async_copy(col_hbm.at[pl.ds(off, CHUNK)], ci[q], dsem[q])
            pltpu.async_copy(w_hbm.at[pl.ds(off, CHUNK)], wv[q], dsem[q])

        def didx_wait(g, q):
            off = ebase + g * CHUNK
            pltpu.make_async_copy(col_hbm.at[pl.ds(off, CHUNK)], ci[q], dsem[q]).wait()
            pltpu.make_async_copy(w_hbm.at[pl.ds(off, CHUNK)], wv[q], dsem[q]).wait()

        pltpu.sync_copy(x_hbm.at[wid], xb_v)
        didx_issue(0, 0)
        didx_issue(1, 1)

        def zrow(j, carry):
            for k in range(dw // LANES):
                wsrc_v[j, pl.ds(k * LANES, LANES)] = jnp.zeros((LANES,), jnp.float32)
            return carry
        lax.fori_loop(0, CHUNK, zrow, 0)
        for kk in range(slab // CHUNK):
            pltpu.sync_copy(wsrc_v, deg_sh.at[pl.ds(sid * slab + kk * CHUNK, CHUNK)])
        plsc.subcore_barrier()

        base = wid * rpt
        bufs = (grow0_v, grow1_v)
        sems = (sem0, sem1)
        pltpu.async_copy(emb_hbm.at[xb_v.at[0]], grow0_v, sem0)
        for i in range(gct):
            if i + 1 < gct:
                pltpu.async_copy(emb_hbm.at[xb_v.at[i + 1]],
                                 bufs[(i + 1) % 2], sems[(i + 1) % 2])
            pltpu.make_async_copy(emb_hbm.at[xb_v.at[i]],
                                  bufs[i % 2], sems[i % 2]).wait()
            pltpu.sync_copy(bufs[i % 2], h0_hbm.at[pl.ds(base + i * GCHUNK, GCHUNK)])

        def deg_pair(sidx, carry):
            for u in range(2):
                g = sidx * 2 + u
                didx_wait(g, u)

                def fill(j, c2):
                    w16 = wv[u][pl.ds(j * LANES, LANES)]
                    for l in range(LANES):
                        wbp = jnp.broadcast_to(w16[l], (LANES,))
                        for k in range(dw // LANES):
                            wsrc_v[j * LANES + l, pl.ds(k * LANES, LANES)] = wbp
                    return c2
                lax.fori_loop(0, CHUNK // LANES, fill, 0)
                pltpu.sync_copy(wsrc_v, deg_sh.at[ci[u]], add=True)

                @pl.when(g + 2 < cpt)
                def _():
                    didx_issue(g + 2, u)
            return carry
        lax.fori_loop(0, cpt // 2, deg_pair, 0)

        plsc.subcore_barrier()
        pltpu.sync_copy(deg_sh.at[pl.ds(sid * slab, slab)],
                        deg_hbm.at[cid, pl.ds(sid * slab, slab)])

    return k1


def _make_hop(F, npad, e2p):
    cpt = e2p // (NC * NS * CHUNK)
    slab = npad // NS
    nf16 = F // LANES

    mesh = plsc.VectorSubcoreMesh(core_axis_name="c", subcore_axis_name="s")

    @functools.partial(
        pl.kernel,
        out_type=jax.ShapeDtypeStruct((NC, npad, F), jnp.float32),
        mesh=mesh,
        scratch_types=[
            pltpu.VMEM((CHUNK,), jnp.int32),
            pltpu.VMEM((CHUNK,), jnp.int32),
            pltpu.VMEM((CHUNK,), jnp.int32),
            pltpu.VMEM((CHUNK,), jnp.int32),
            pltpu.VMEM((CHUNK,), jnp.int32),
            pltpu.VMEM((CHUNK,), jnp.int32),
            pltpu.VMEM((CHUNK,), jnp.float32),
            pltpu.VMEM((CHUNK,), jnp.float32),
            pltpu.VMEM((CHUNK,), jnp.float32),
            pltpu.VMEM((CHUNK, F), jnp.float32),
            pltpu.VMEM((CHUNK, F), jnp.float32),
            pltpu.VMEM_SHARED((npad, F), jnp.float32),
            pltpu.SemaphoreType.DMA,
            pltpu.SemaphoreType.DMA,
            pltpu.SemaphoreType.DMA,
            pltpu.SemaphoreType.DMA,
            pltpu.SemaphoreType.DMA,
        ],
    )
    def hop(t_hbm, row_hbm, col_hbm, w_hbm, part_hbm,
            ri0_v, ri1_v, ri2_v, ci0_v, ci1_v, ci2_v, wv0_v, wv1_v, wv2_v,
            rows0_v, rows1_v, acc_sh, is0, is1, is2, gs0, gs1):
        cid = lax.axis_index("c")
        sid = lax.axis_index("s")
        wid = cid * NS + sid
        ri = (ri0_v, ri1_v, ri2_v)
        ci = (ci0_v, ci1_v, ci2_v)
        wv = (wv0_v, wv1_v, wv2_v)
        rows = (rows0_v, rows1_v)
        isem = (is0, is1, is2)
        gsem = (gs0, gs1)
        ebase = wid * cpt * CHUNK

        def idx_issue(g, q):
            off = ebase + g * CHUNK
            pltpu.async_copy(row_hbm.at[pl.ds(off, CHUNK)], ri[q], isem[q])
            pltpu.async_copy(col_hbm.at[pl.ds(off, CHUNK)], ci[q], isem[q])
            pltpu.async_copy(w_hbm.at[pl.ds(off, CHUNK)], wv[q], isem[q])

        def idx_wait(g, q):
            off = ebase + g * CHUNK
            pltpu.make_async_copy(row_hbm.at[pl.ds(off, CHUNK)], ri[q], isem[q]).wait()
            pltpu.make_async_copy(col_hbm.at[pl.ds(off, CHUNK)], ci[q], isem[q]).wait()
            pltpu.make_async_copy(w_hbm.at[pl.ds(off, CHUNK)], wv[q], isem[q]).wait()

        idx_issue(0, 0)
        idx_issue(1, 1)
        idx_issue(2, 2)

        def zrow(j, carry):
            for k in range(nf16):
                rows0_v[j, pl.ds(k * LANES, LANES)] = jnp.zeros((LANES,), jnp.float32)
            return carry
        lax.fori_loop(0, CHUNK, zrow, 0)
        for kk in range(slab // CHUNK):
            pltpu.sync_copy(rows0_v, acc_sh.at[pl.ds(sid * slab + kk * CHUNK, CHUNK)])
        plsc.subcore_barrier()

        idx_wait(0, 0)
        pltpu.async_copy(t_hbm.at[ri[0]], rows0_v, gs0)
        idx_wait(1, 1)
        pltpu.async_copy(t_hbm.at[ri[1]], rows1_v, gs1)

        def six_body(sidx, carry):
            g0 = sidx * 6
            for u in range(6):
                g = g0 + u
                P = u % 2
                Q = u % 3
                Q2 = (u + 2) % 3
                pltpu.make_async_copy(t_hbm.at[ri[Q]], rows[P], gsem[P]).wait()

                def scale(j, c2):
                    w16 = wv[Q][pl.ds(j * LANES, LANES)]
                    for l in range(LANES):
                        e = j * LANES + l
                        s = w16[l]
                        for k in range(nf16):
                            rows[P][e, pl.ds(k * LANES, LANES)] = (
                                rows[P][e, pl.ds(k * LANES, LANES)] * s)
                    return c2
                lax.fori_loop(0, CHUNK // LANES, scale, 0)
                pltpu.sync_copy(rows[P], acc_sh.at[ci[Q]], add=True)

                @pl.when(g + 3 < cpt)
                def _():
                    idx_issue(g + 3, Q)

                @pl.when(g + 2 < cpt)
                def _():
                    idx_wait(g + 2, Q2)
                    pltpu.async_copy(t_hbm.at[ri[Q2]], rows[P], gsem[P])
            return carry
        lax.fori_loop(0, cpt // 6, six_body, 0)

        plsc.subcore_barrier()
        pltpu.sync_copy(acc_sh.at[pl.ds(sid * slab, slab)],
                        part_hbm.at[cid, pl.ds(sid * slab, slab)])

    return hop


def _prescale(deg, h0, npad, F, dw):
    grid = npad // ROWBLK

    def body(deg_ref, h0_ref, t0_ref, dis_ref):
        d = deg_ref[0] + deg_ref[1]
        degs = jnp.sum(d, axis=1, keepdims=True) * (1.0 / dw)
        dis = jnp.where(degs > 0, lax.rsqrt(degs), 0.0)
        disb = jnp.broadcast_to(dis, (ROWBLK, F))
        dis_ref[...] = disb
        t0_ref[...] = h0_ref[...] * disb

    return pl.pallas_call(
        body,
        grid=(grid,),
        in_specs=[
            pl.BlockSpec((NC, ROWBLK, dw), lambda i: (0, i, 0)),
            pl.BlockSpec((ROWBLK, F), lambda i: (i, 0)),
        ],
        out_specs=[
            pl.BlockSpec((ROWBLK, F), lambda i: (i, 0)),
            pl.BlockSpec((ROWBLK, F), lambda i: (i, 0)),
        ],
        out_shape=[
            jax.ShapeDtypeStruct((npad, F), jnp.float32),
            jax.ShapeDtypeStruct((npad, F), jnp.float32),
        ],
    )(deg, h0)


def _midscale(part, dis, npad, F):
    grid = npad // ROWBLK

    def body(p_ref, dis_ref, o_ref):
        d = dis_ref[...]
        o_ref[...] = (p_ref[0] + p_ref[1]) * d * d

    return pl.pallas_call(
        body,
        grid=(grid,),
        in_specs=[
            pl.BlockSpec((NC, ROWBLK, F), lambda i: (0, i, 0)),
            pl.BlockSpec((ROWBLK, F), lambda i: (i, 0)),
        ],
        out_specs=pl.BlockSpec((ROWBLK, F), lambda i: (i, 0)),
        out_shape=jax.ShapeDtypeStruct((npad, F), jnp.float32),
    )(part, dis)


def _final(part, dis, W, b2, n, npad, F, C):
    grid = npad // ROWBLK

    def body(p_ref, dis_ref, w_ref, b_ref, o_ref):
        h = (p_ref[0] + p_ref[1]) * dis_ref[...]
        z = jnp.dot(h, w_ref[...], preferred_element_type=jnp.float32)
        z = z + b_ref[...]
        m = jnp.max(z, axis=1, keepdims=True)
        e = jnp.exp(z - m)
        lse = jnp.log(jnp.sum(e, axis=1, keepdims=True)) + m
        o_ref[...] = z - lse

    return pl.pallas_call(
        body,
        grid=(grid,),
        in_specs=[
            pl.BlockSpec((NC, ROWBLK, F), lambda i: (0, i, 0)),
            pl.BlockSpec((ROWBLK, F), lambda i: (i, 0)),
            pl.BlockSpec((F, C), lambda i: (0, 0)),
            pl.BlockSpec((1, C), lambda i: (0, 0)),
        ],
        out_specs=pl.BlockSpec((ROWBLK, C), lambda i: (i, 0)),
        out_shape=jax.ShapeDtypeStruct((n, C), jnp.float32),
    )(part, dis, W, b2)


def kernel(x, edge_index, edge_attr, emb, W, b):
    n = x.shape[0]
    F = emb.shape[1]
    C = W.shape[1]
    E = edge_attr.shape[0]
    dw = F

    npad = _round_up(n, NC * NS * GCHUNK)
    e2 = E + n
    e2p = _round_up(e2, NC * NS * CHUNK * 6)

    idt = jnp.int32
    loop_idx = jnp.arange(n, dtype=idt)
    row2 = jnp.concatenate([edge_index[0].astype(idt), loop_idx])
    col2 = jnp.concatenate([edge_index[1].astype(idt), loop_idx])
    w2 = jnp.concatenate([edge_attr.astype(jnp.float32),
                          jnp.ones((n,), jnp.float32)])
    pad = e2p - e2
    gct = npad // (NC * NS * GCHUNK)
    row2 = jnp.pad(row2, (0, pad))
    col2 = jnp.pad(col2, (0, pad))
    w2 = jnp.pad(w2, (0, pad))
    xp = jnp.pad(x.astype(idt), (0, npad - n)).reshape(NC * NS, gct, GCHUNK)

    h0, deg = _make_gather_deg(F, npad, e2p, dw)(emb, xp, col2, w2)
    t0, dis = _prescale(deg, h0, npad, F, dw)
    hop = _make_hop(F, npad, e2p)
    p1 = hop(t0, row2, col2, w2)
    t1 = _midscale(p1, dis, npad, F)
    p2 = hop(t1, row2, col2, w2)
    return _final(p2, dis, W, b.reshape(1, C), n, npad, F, C)


# speedup vs baseline: 1.1350x; 1.1350x over previous
import functools

import jax
import jax.numpy as jnp
from jax import lax
from jax.experimental import pallas as pl
from jax.experimental.pallas import tpu as pltpu
from jax.experimental.pallas import tpu_sc as plsc

NC = 2
NS = 16
LANES = 16
CHUNK = 128
GCHUNK = 64
ROWBLK = 1024


def _round_up(v, m):
    return (v + m - 1) // m * m


def _make_gather_deg(F, npad, e2p, dw):
    cpt = e2p // (NC * NS * CHUNK)
    rpt = npad // (NC * NS)
    gct = rpt // GCHUNK
    slab = npad // NS

    mesh = plsc.VectorSubcoreMesh(core_axis_name="c", subcore_axis_name="s")

    @functools.partial(
        pl.kernel,
        out_type=(
            jax.ShapeDtypeStruct((npad, F), jnp.float32),
            jax.ShapeDtypeStruct((NC, npad, dw), jnp.float32),
        ),
        mesh=mesh,
        scratch_types=[
            pltpu.VMEM((rpt // GCHUNK, GCHUNK), jnp.int32),
            pltpu.VMEM((GCHUNK, F), jnp.float32),
            pltpu.VMEM((GCHUNK, F), jnp.float32),
            pltpu.VMEM((2, CHUNK), jnp.int32),
            pltpu.VMEM((2, CHUNK), jnp.int32),
            pltpu.VMEM((1, CHUNK), jnp.float32),
            pltpu.VMEM((1, CHUNK), jnp.float32),
            pltpu.VMEM((CHUNK, dw), jnp.float32),
            pltpu.VMEM_SHARED((npad, dw), jnp.float32),
            pltpu.SemaphoreType.DMA,
            pltpu.SemaphoreType.DMA,
            pltpu.SemaphoreType.DMA,
            pltpu.SemaphoreType.DMA,
        ],
    )
    def k1(emb_hbm, x_hbm, ed_hbm, wd_hbm, h0_hbm, deg_hbm,
           xb_v, grow0_v, grow1_v, eb0_v, eb1_v, wv0_v, wv1_v, wsrc_v, deg_sh,
           sem0, sem1, dsem0, dsem1):
        cid = lax.axis_index("c")
        sid = lax.axis_index("s")
        wid = cid * NS + sid
        eb = (eb0_v, eb1_v)
        wv = (wv0_v, wv1_v)
        dsem = (dsem0, dsem1)
        cbase = wid * cpt

        def didx_issue(g, q):
            pltpu.async_copy(ed_hbm.at[cbase + g], eb[q], dsem[q])
            pltpu.async_copy(wd_hbm.at[cbase + g], wv[q], dsem[q])

        def didx_wait(g, q):
            pltpu.make_async_copy(ed_hbm.at[cbase + g], eb[q], dsem[q]).wait()
            pltpu.make_async_copy(wd_hbm.at[cbase + g], wv[q], dsem[q]).wait()

        pltpu.sync_copy(x_hbm.at[wid], xb_v)
        didx_issue(0, 0)
        didx_issue(1, 1)

        def zrow(j, carry):
            for k in range(dw // LANES):
                wsrc_v[j, pl.ds(k * LANES, LANES)] = jnp.zeros((LANES,), jnp.float32)
            return carry
        lax.fori_loop(0, CHUNK, zrow, 0)
        for kk in range(slab // CHUNK):
            pltpu.sync_copy(wsrc_v, deg_sh.at[pl.ds(sid * slab + kk * CHUNK, CHUNK)])
        plsc.subcore_barrier()

        base = wid * rpt
        bufs = (grow0_v, grow1_v)
        sems = (sem0, sem1)
        pltpu.async_copy(emb_hbm.at[xb_v.at[0]], grow0_v, sem0)
        for i in range(gct):
            if i + 1 < gct:
                pltpu.async_copy(emb_hbm.at[xb_v.at[i + 1]],
                                 bufs[(i + 1) % 2], sems[(i + 1) % 2])
            pltpu.make_async_copy(emb_hbm.at[xb_v.at[i]],
                                  bufs[i % 2], sems[i % 2]).wait()
            pltpu.sync_copy(bufs[i % 2], h0_hbm.at[pl.ds(base + i * GCHUNK, GCHUNK)])

        def deg_pair(sidx, carry):
            for u in range(2):
                g = sidx * 2 + u
                didx_wait(g, u)

                def fill(j, c2):
                    w16 = wv[u][0, pl.ds(j * LANES, LANES)]
                    for l in range(LANES):
                        wbp = jnp.broadcast_to(w16[l], (LANES,))
                        for k in range(dw // LANES):
                            wsrc_v[j * LANES + l, pl.ds(k * LANES, LANES)] = wbp
                    return c2
                lax.fori_loop(0, CHUNK // LANES, fill, 0)
                pltpu.sync_copy(wsrc_v, deg_sh.at[eb[u].at[1]], add=True)

                @pl.when(g + 2 < cpt)
                def _():
                    didx_issue(g + 2, u)
            return carry
        lax.fori_loop(0, cpt // 2, deg_pair, 0)

        plsc.subcore_barrier()
        pltpu.sync_copy(deg_sh.at[pl.ds(sid * slab, slab)],
                        deg_hbm.at[cid, pl.ds(sid * slab, slab)])

    return k1


def _make_hop(F, npad, e2p, nfs):
    cpt = e2p // (NC * NS * CHUNK)
    slab = npad // NS
    nf16 = F // LANES

    mesh = plsc.VectorSubcoreMesh(core_axis_name="c", subcore_axis_name="s")

    @functools.partial(
        pl.kernel,
        out_type=jax.ShapeDtypeStruct((NC, npad, F), jnp.float32),
        mesh=mesh,
        scratch_types=[
            pltpu.VMEM((2, CHUNK), jnp.int32),
            pltpu.VMEM((2, CHUNK), jnp.int32),
            pltpu.VMEM((1, CHUNK), jnp.float32),
            pltpu.VMEM((1, CHUNK), jnp.float32),
            pltpu.VMEM((CHUNK, F), jnp.float32),
            pltpu.VMEM((CHUNK, F), jnp.float32),
            pltpu.VMEM_SHARED((npad, F), jnp.float32),
            pltpu.SemaphoreType.DMA,
            pltpu.SemaphoreType.DMA,
        ],
    )
    def hop(t_hbm, ed_hbm, wd_hbm, part_hbm,
            eb0_v, eb1_v, wv0_v, wv1_v, rows0_v, rows1_v, acc_sh, gs0, gs1):
        cid = lax.axis_index("c")
        sid = lax.axis_index("s")
        wid = cid * NS + sid
        eb = (eb0_v, eb1_v)
        wv = (wv0_v, wv1_v)
        rows = (rows0_v, rows1_v)
        gsem = (gs0, gs1)
        cbase = wid * cpt

        def zrow(j, carry):
            for k in range(nf16):
                rows0_v[j, pl.ds(k * LANES, LANES)] = (
                    jnp.zeros((LANES,), jnp.float32))
            return carry
        lax.fori_loop(0, CHUNK, zrow, 0)
        for kk in range(slab // CHUNK):
            pltpu.sync_copy(rows0_v,
                            acc_sh.at[pl.ds(sid * slab + kk * CHUNK, CHUNK)])
        plsc.subcore_barrier()

        pltpu.sync_copy(ed_hbm.at[cbase], eb0_v)
        pltpu.sync_copy(wd_hbm.at[cbase], wv0_v)
        pltpu.async_copy(t_hbm.at[eb0_v.at[0]], rows0_v, gs0)

        def pair_body(sidx, carry):
            for u in range(2):
                g = sidx * 2 + u
                o = u ^ 1

                @pl.when(g + 1 < cpt)
                def _():
                    pltpu.sync_copy(ed_hbm.at[cbase + g + 1], eb[o])
                    pltpu.sync_copy(wd_hbm.at[cbase + g + 1], wv[o])
                    pltpu.async_copy(t_hbm.at[eb[o].at[0]], rows[o], gsem[o])
                pltpu.make_async_copy(t_hbm.at[eb[u].at[0]], rows[u],
                                      gsem[u]).wait()

                def scale(j, c2):
                    w16 = wv[u][0, pl.ds(j * LANES, LANES)]
                    for l in range(LANES):
                        e = j * LANES + l
                        sc = w16[l]
                        for k in range(nfs):
                            rows[u][e, pl.ds(k * LANES, LANES)] = (
                                rows[u][e, pl.ds(k * LANES, LANES)] * sc)
                    return c2
                lax.fori_loop(0, CHUNK // LANES, scale, 0)
                pltpu.sync_copy(rows[u], acc_sh.at[eb[u].at[1]], add=True)
            return carry
        lax.fori_loop(0, cpt // 2, pair_body, 0)

        plsc.subcore_barrier()
        pltpu.sync_copy(acc_sh.at[pl.ds(sid * slab, slab)],
                        part_hbm.at[cid, pl.ds(sid * slab, slab)])

    return hop


def _prescale(deg, h0, W, npad, F, C, dw):
    grid = npad // ROWBLK

    def body(deg_ref, h0_ref, w_ref, t0_ref, dis_ref):
        d = deg_ref[0] + deg_ref[1]
        degs = jnp.sum(d, axis=1, keepdims=True) * (1.0 / dw)
        dis = jnp.where(degs > 0, lax.rsqrt(degs), 0.0)
        disb = jnp.broadcast_to(dis, (ROWBLK, C))
        dis_ref[...] = disb
        y = jnp.dot(h0_ref[...], w_ref[...],
                    preferred_element_type=jnp.float32)
        t0_ref[...] = jnp.concatenate(
            [y * disb, jnp.zeros((ROWBLK, F - C), jnp.float32)], axis=1)

    return pl.pallas_call(
        body,
        grid=(grid,),
        in_specs=[
            pl.BlockSpec((NC, ROWBLK, dw), lambda i: (0, i, 0)),
            pl.BlockSpec((ROWBLK, F), lambda i: (i, 0)),
            pl.BlockSpec((F, C), lambda i: (0, 0)),
        ],
        out_specs=[
            pl.BlockSpec((ROWBLK, F), lambda i: (i, 0)),
            pl.BlockSpec((ROWBLK, C), lambda i: (i, 0)),
        ],
        out_shape=[
            jax.ShapeDtypeStruct((npad, F), jnp.float32),
            jax.ShapeDtypeStruct((npad, C), jnp.float32),
        ],
    )(deg, h0, W)


def _midscale(part, dis, npad, F, C):
    grid = npad // ROWBLK

    def body(p_ref, dis_ref, o_ref):
        d = dis_ref[...]
        o_ref[...] = jnp.concatenate(
            [(p_ref[0] + p_ref[1])[:, :C] * d * d,
             jnp.zeros((ROWBLK, F - C), jnp.float32)], axis=1)

    return pl.pallas_call(
        body,
        grid=(grid,),
        in_specs=[
            pl.BlockSpec((NC, ROWBLK, F), lambda i: (0, i, 0)),
            pl.BlockSpec((ROWBLK, C), lambda i: (i, 0)),
        ],
        out_specs=pl.BlockSpec((ROWBLK, F), lambda i: (i, 0)),
        out_shape=jax.ShapeDtypeStruct((npad, F), jnp.float32),
    )(part, dis)


def _final(part, dis, b2, n, npad, F, C):
    grid = npad // ROWBLK

    def body(p_ref, dis_ref, b_ref, o_ref):
        z = (p_ref[0] + p_ref[1])[:, :C] * dis_ref[...] + b_ref[...]
        m = jnp.max(z, axis=1, keepdims=True)
        e = jnp.exp(z - m)
        lse = jnp.log(jnp.sum(e, axis=1, keepdims=True)) + m
        o_ref[...] = z - lse

    return pl.pallas_call(
        body,
        grid=(grid,),
        in_specs=[
            pl.BlockSpec((NC, ROWBLK, F), lambda i: (0, i, 0)),
            pl.BlockSpec((ROWBLK, C), lambda i: (i, 0)),
            pl.BlockSpec((1, C), lambda i: (0, 0)),
        ],
        out_specs=pl.BlockSpec((ROWBLK, C), lambda i: (i, 0)),
        out_shape=jax.ShapeDtypeStruct((n, C), jnp.float32),
    )(part, dis, b2)


def kernel(x, edge_index, edge_attr, emb, W, b):
    n = x.shape[0]
    F = emb.shape[1]
    C = W.shape[1]
    E = edge_attr.shape[0]
    dw = F

    npad = _round_up(n, NC * NS * GCHUNK)
    e2 = E + n
    e2p = _round_up(e2, NC * NS * CHUNK * 6)

    idt = jnp.int32
    loop_idx = jnp.arange(n, dtype=idt)
    row2 = jnp.concatenate([edge_index[0].astype(idt), loop_idx])
    col2 = jnp.concatenate([edge_index[1].astype(idt), loop_idx])
    w2 = jnp.concatenate([edge_attr.astype(jnp.float32),
                          jnp.ones((n,), jnp.float32)])
    pad = e2p - e2
    gct = npad // (NC * NS * GCHUNK)
    row2 = jnp.pad(row2, (0, pad)).reshape(e2p // CHUNK, 1, CHUNK)
    col2 = jnp.pad(col2, (0, pad)).reshape(e2p // CHUNK, 1, CHUNK)
    edata = jnp.concatenate([row2, col2], axis=1)
    wdata = jnp.pad(w2, (0, pad)).reshape(e2p // CHUNK, 1, CHUNK)
    xp = jnp.pad(x.astype(idt), (0, npad - n)).reshape(NC * NS, gct, GCHUNK)

    h0, deg = _make_gather_deg(F, npad, e2p, dw)(emb, xp, edata, wdata)
    t0, dis = _prescale(deg, h0, W, npad, F, C, dw)
    hop = _make_hop(F, npad, e2p, C // LANES)
    p1 = hop(t0, edata, wdata)
    t1 = _midscale(p1, dis, npad, F, C)
    p2 = hop(t1, edata, wdata)
    return _final(p2, dis, b.reshape(1, C), n, npad, F, C)


# speedup vs baseline: 1.2128x; 1.0686x over previous
import functools

import jax
import jax.numpy as jnp
from jax import lax
from jax.experimental import pallas as pl
from jax.experimental.pallas import tpu as pltpu
from jax.experimental.pallas import tpu_sc as plsc

NC = 2
NS = 16
LANES = 16
CHUNK = 128
GCHUNK = 64
ROWBLK = 1024


def _round_up(v, m):
    return (v + m - 1) // m * m


def _make_gather_deg(F, npad, e2p, dw):
    cpt = e2p // (NC * NS * CHUNK)
    rpt = npad // (NC * NS)
    gct = rpt // GCHUNK
    slab = npad // NS

    mesh = plsc.VectorSubcoreMesh(core_axis_name="c", subcore_axis_name="s")

    @functools.partial(
        pl.kernel,
        out_type=(
            jax.ShapeDtypeStruct((npad, F), jnp.float32),
            jax.ShapeDtypeStruct((NC, npad, dw), jnp.float32),
        ),
        mesh=mesh,
        scratch_types=[
            pltpu.VMEM((rpt // GCHUNK, GCHUNK), jnp.int32),
            pltpu.VMEM((GCHUNK, F), jnp.float32),
            pltpu.VMEM((GCHUNK, F), jnp.float32),
            pltpu.VMEM((2, CHUNK), jnp.int32),
            pltpu.VMEM((2, CHUNK), jnp.int32),
            pltpu.VMEM((1, CHUNK), jnp.float32),
            pltpu.VMEM((1, CHUNK), jnp.float32),
            pltpu.VMEM((CHUNK, dw), jnp.float32),
            pltpu.VMEM_SHARED((npad, dw), jnp.float32),
            pltpu.SemaphoreType.DMA,
            pltpu.SemaphoreType.DMA,
            pltpu.SemaphoreType.DMA,
            pltpu.SemaphoreType.DMA,
        ],
    )
    def k1(emb_hbm, x_hbm, ed_hbm, wd_hbm, h0_hbm, deg_hbm,
           xb_v, grow0_v, grow1_v, eb0_v, eb1_v, wv0_v, wv1_v, wsrc_v, deg_sh,
           sem0, sem1, dsem0, dsem1):
        cid = lax.axis_index("c")
        sid = lax.axis_index("s")
        wid = cid * NS + sid
        eb = (eb0_v, eb1_v)
        wv = (wv0_v, wv1_v)
        dsem = (dsem0, dsem1)
        cbase = wid * cpt

        def didx_issue(g, q):
            pltpu.async_copy(ed_hbm.at[cbase + g], eb[q], dsem[q])
            pltpu.async_copy(wd_hbm.at[cbase + g], wv[q], dsem[q])

        def didx_wait(g, q):
            pltpu.make_async_copy(ed_hbm.at[cbase + g], eb[q], dsem[q]).wait()
            pltpu.make_async_copy(wd_hbm.at[cbase + g], wv[q], dsem[q]).wait()

        pltpu.sync_copy(x_hbm.at[wid], xb_v)
        didx_issue(0, 0)
        didx_issue(1, 1)

        def zrow(j, carry):
            for k in range(dw // LANES):
                wsrc_v[j, pl.ds(k * LANES, LANES)] = jnp.zeros((LANES,), jnp.float32)
            return carry
        lax.fori_loop(0, CHUNK, zrow, 0)
        for kk in range(slab // CHUNK):
            pltpu.sync_copy(wsrc_v, deg_sh.at[pl.ds(sid * slab + kk * CHUNK, CHUNK)])
        plsc.subcore_barrier()

        base = wid * rpt
        bufs = (grow0_v, grow1_v)
        sems = (sem0, sem1)
        pltpu.async_copy(emb_hbm.at[xb_v.at[0]], grow0_v, sem0)
        for i in range(gct):
            if i + 1 < gct:
                pltpu.async_copy(emb_hbm.at[xb_v.at[i + 1]],
                                 bufs[(i + 1) % 2], sems[(i + 1) % 2])
            pltpu.make_async_copy(emb_hbm.at[xb_v.at[i]],
                                  bufs[i % 2], sems[i % 2]).wait()
            pltpu.sync_copy(bufs[i % 2], h0_hbm.at[pl.ds(base + i * GCHUNK, GCHUNK)])

        def deg_pair(sidx, carry):
            for u in range(2):
                g = sidx * 2 + u
                didx_wait(g, u)

                def fill(j, c2):
                    w16 = wv[u][0, pl.ds(j * LANES, LANES)]
                    for l in range(LANES):
                        wbp = jnp.broadcast_to(w16[l], (LANES,))
                        for k in range(dw // LANES):
                            wsrc_v[j * LANES + l, pl.ds(k * LANES, LANES)] = wbp
                    return c2
                lax.fori_loop(0, CHUNK // LANES, fill, 0)
                pltpu.sync_copy(wsrc_v, deg_sh.at[eb[u].at[1]], add=True)

                @pl.when(g + 2 < cpt)
                def _():
                    didx_issue(g + 2, u)
            return carry
        lax.fori_loop(0, cpt // 2, deg_pair, 0)

        plsc.subcore_barrier()
        pltpu.sync_copy(deg_sh.at[pl.ds(sid * slab, slab)],
                        deg_hbm.at[cid, pl.ds(sid * slab, slab)])

    return k1


def _make_hop(F, npad, e2p):
    cpt = e2p // (NC * NS * CHUNK)
    slab = npad // NS
    nf16 = F // LANES

    mesh = plsc.VectorSubcoreMesh(core_axis_name="c", subcore_axis_name="s")

    @functools.partial(
        pl.kernel,
        out_type=jax.ShapeDtypeStruct((NC, npad, F), jnp.float32),
        mesh=mesh,
        scratch_types=[
            pltpu.VMEM((2, CHUNK), jnp.int32),
            pltpu.VMEM((2, CHUNK), jnp.int32),
            pltpu.VMEM((1, CHUNK), jnp.float32),
            pltpu.VMEM((1, CHUNK), jnp.float32),
            pltpu.VMEM((CHUNK, F), jnp.float32),
            pltpu.VMEM((CHUNK, F), jnp.float32),
            pltpu.VMEM_SHARED((npad, F), jnp.float32),
            pltpu.SemaphoreType.DMA,
            pltpu.SemaphoreType.DMA,
        ],
    )
    def hop(t_hbm, ed_hbm, wd_hbm, part_hbm,
            eb0_v, eb1_v, wv0_v, wv1_v, rows0_v, rows1_v, acc_sh, gs0, gs1):
        cid = lax.axis_index("c")
        sid = lax.axis_index("s")
        wid = cid * NS + sid
        eb = (eb0_v, eb1_v)
        wv = (wv0_v, wv1_v)
        rows = (rows0_v, rows1_v)
        gsem = (gs0, gs1)
        cbase = wid * cpt

        def zrow(j, carry):
            for k in range(nf16):
                rows0_v[j, pl.ds(k * LANES, LANES)] = (
                    jnp.zeros((LANES,), jnp.float32))
            return carry
        lax.fori_loop(0, CHUNK, zrow, 0)
        for kk in range(slab // CHUNK):
            pltpu.sync_copy(rows0_v,
                            acc_sh.at[pl.ds(sid * slab + kk * CHUNK, CHUNK)])
        plsc.subcore_barrier()

        pltpu.sync_copy(ed_hbm.at[cbase], eb0_v)
        pltpu.sync_copy(wd_hbm.at[cbase], wv0_v)
        pltpu.async_copy(t_hbm.at[eb0_v.at[0]], rows0_v, gs0)

        def pair_body(sidx, carry):
            for u in range(2):
                g = sidx * 2 + u
                o = u ^ 1

                @pl.when(g + 1 < cpt)
                def _():
                    pltpu.sync_copy(ed_hbm.at[cbase + g + 1], eb[o])
                    pltpu.sync_copy(wd_hbm.at[cbase + g + 1], wv[o])
                    pltpu.async_copy(t_hbm.at[eb[o].at[0]], rows[o], gsem[o])
                pltpu.make_async_copy(t_hbm.at[eb[u].at[0]], rows[u],
                                      gsem[u]).wait()

                def scale(j, c2):
                    w16 = wv[u][0, pl.ds(j * LANES, LANES)]
                    for l in range(LANES):
                        e = j * LANES + l
                        sc = w16[l]
                        for k in range(nf16):
                            rows[u][e, pl.ds(k * LANES, LANES)] = (
                                rows[u][e, pl.ds(k * LANES, LANES)] * sc)
                    return c2
                lax.fori_loop(0, CHUNK // LANES, scale, 0)
                pltpu.sync_copy(rows[u], acc_sh.at[eb[u].at[1]], add=True)
            return carry
        lax.fori_loop(0, cpt // 2, pair_body, 0)

        plsc.subcore_barrier()
        pltpu.sync_copy(acc_sh.at[pl.ds(sid * slab, slab)],
                        part_hbm.at[cid, pl.ds(sid * slab, slab)])

    return hop


def _prescale(deg, h0, npad, F, dw):
    grid = npad // ROWBLK

    def body(deg_ref, h0_ref, t0_ref, dis_ref):
        d = deg_ref[0] + deg_ref[1]
        degs = jnp.sum(d, axis=1, keepdims=True) * (1.0 / dw)
        dis = jnp.where(degs > 0, lax.rsqrt(degs), 0.0)
        disb = jnp.broadcast_to(dis, (ROWBLK, F))
        dis_ref[...] = disb
        t0_ref[...] = h0_ref[...] * disb

    return pl.pallas_call(
        body,
        grid=(grid,),
        in_specs=[
            pl.BlockSpec((NC, ROWBLK, dw), lambda i: (0, i, 0)),
            pl.BlockSpec((ROWBLK, F), lambda i: (i, 0)),
        ],
        out_specs=[
            pl.BlockSpec((ROWBLK, F), lambda i: (i, 0)),
            pl.BlockSpec((ROWBLK, F), lambda i: (i, 0)),
        ],
        out_shape=[
            jax.ShapeDtypeStruct((npad, F), jnp.float32),
            jax.ShapeDtypeStruct((npad, F), jnp.float32),
        ],
    )(deg, h0)


def _midscale(part, dis, npad, F):
    grid = npad // ROWBLK

    def body(p_ref, dis_ref, o_ref):
        d = dis_ref[...]
        o_ref[...] = (p_ref[0] + p_ref[1]) * d * d

    return pl.pallas_call(
        body,
        grid=(grid,),
        in_specs=[
            pl.BlockSpec((NC, ROWBLK, F), lambda i: (0, i, 0)),
            pl.BlockSpec((ROWBLK, F), lambda i: (i, 0)),
        ],
        out_specs=pl.BlockSpec((ROWBLK, F), lambda i: (i, 0)),
        out_shape=jax.ShapeDtypeStruct((npad, F), jnp.float32),
    )(part, dis)


def _final(part, dis, W, b2, n, npad, F, C):
    grid = npad // ROWBLK

    def body(p_ref, dis_ref, w_ref, b_ref, o_ref):
        h = (p_ref[0] + p_ref[1]) * dis_ref[...]
        z = jnp.dot(h, w_ref[...], preferred_element_type=jnp.float32)
        z = z + b_ref[...]
        m = jnp.max(z, axis=1, keepdims=True)
        e = jnp.exp(z - m)
        lse = jnp.log(jnp.sum(e, axis=1, keepdims=True)) + m
        o_ref[...] = z - lse

    return pl.pallas_call(
        body,
        grid=(grid,),
        in_specs=[
            pl.BlockSpec((NC, ROWBLK, F), lambda i: (0, i, 0)),
            pl.BlockSpec((ROWBLK, F), lambda i: (i, 0)),
            pl.BlockSpec((F, C), lambda i: (0, 0)),
            pl.BlockSpec((1, C), lambda i: (0, 0)),
        ],
        out_specs=pl.BlockSpec((ROWBLK, C), lambda i: (i, 0)),
        out_shape=jax.ShapeDtypeStruct((n, C), jnp.float32),
    )(part, dis, W, b2)


def kernel(x, edge_index, edge_attr, emb, W, b):
    n = x.shape[0]
    F = emb.shape[1]
    C = W.shape[1]
    E = edge_attr.shape[0]
    dw = F

    npad = _round_up(n, NC * NS * GCHUNK)
    e2 = E + n
    e2p = _round_up(e2, NC * NS * CHUNK * 6)

    idt = jnp.int32
    loop_idx = jnp.arange(n, dtype=idt)
    row2 = jnp.concatenate([edge_index[0].astype(idt), loop_idx])
    col2 = jnp.concatenate([edge_index[1].astype(idt), loop_idx])
    w2 = jnp.concatenate([edge_attr.astype(jnp.float32),
                          jnp.ones((n,), jnp.float32)])
    pad = e2p - e2
    gct = npad // (NC * NS * GCHUNK)
    row2 = jnp.pad(row2, (0, pad)).reshape(e2p // CHUNK, 1, CHUNK)
    col2 = jnp.pad(col2, (0, pad)).reshape(e2p // CHUNK, 1, CHUNK)
    edata = jnp.concatenate([row2, col2], axis=1)
    wdata = jnp.pad(w2, (0, pad)).reshape(e2p // CHUNK, 1, CHUNK)
    xp = jnp.pad(x.astype(idt), (0, npad - n)).reshape(NC * NS, gct, GCHUNK)

    h0, deg = _make_gather_deg(F, npad, e2p, dw)(emb, xp, edata, wdata)
    t0, dis = _prescale(deg, h0, npad, F, dw)
    hop = _make_hop(F, npad, e2p)
    p1 = hop(t0, edata, wdata)
    t1 = _midscale(p1, dis, npad, F)
    p2 = hop(t1, edata, wdata)
    return _final(p2, dis, W, b.reshape(1, C), n, npad, F, C)
